# baseline (speedup 1.0000x reference)
import functools

import jax
import jax.numpy as jnp
from jax import lax
from jax.experimental import pallas as pl
from jax.experimental.pallas import tpu as pltpu
from jax.experimental.pallas import tpu_sc as plsc

_TOPK = 8
_E = 64
_H = 2048
_LANES = 16


def _softmax64(logits):
    m = jnp.max(logits, axis=-1, keepdims=True)
    e = jnp.exp(logits - m)
    return e / jnp.sum(e, axis=-1, keepdims=True)


def _probs_body(xa_ref, xb_ref, wt_ref, packed_ref):
    pa = _softmax64(
        jnp.dot(xa_ref[...], wt_ref[...], preferred_element_type=jnp.float32)
    )
    pb = _softmax64(
        jnp.dot(xb_ref[...], wt_ref[...], preferred_element_type=jnp.float32)
    )
    packed_ref[...] = jnp.concatenate([pa, pb], axis=1)


def _router_probs(x, wt, block_rows=1024):
    n = x.shape[0]
    half = n // block_rows // 2
    return pl.pallas_call(
        _probs_body,
        grid=(half,),
        in_specs=[
            pl.BlockSpec((block_rows, _H), lambda i: (i, 0)),
            pl.BlockSpec((block_rows, _H), lambda i: (i + half, 0)),
            pl.BlockSpec((_H, _E), lambda i: (0, 0)),
        ],
        out_specs=pl.BlockSpec((block_rows, 2 * _E), lambda i: (i, 0)),
        out_shape=jax.ShapeDtypeStruct((n // 2, 2 * _E), jnp.float32),
    )(x, x, wt)


def _make_topk(n_rows):
    info = plsc.get_sparse_core_info()
    nc, ns = info.num_cores, info.num_subcores
    nw = nc * ns
    rows_per_w = n_rows // nw

    mesh = plsc.VectorSubcoreMesh(core_axis_name="c", subcore_axis_name="s")

    @functools.partial(
        pl.kernel,
        mesh=mesh,
        out_type=(
            jax.ShapeDtypeStruct((n_rows * _TOPK,), jnp.float32),
            jax.ShapeDtypeStruct((n_rows * _TOPK,), jnp.int32),
        ),
        scratch_types=[
            pltpu.VMEM((rows_per_w, _E), jnp.float32),
            pltpu.VMEM((rows_per_w * _TOPK,), jnp.float32),
            pltpu.VMEM((rows_per_w * _TOPK,), jnp.int32),
        ],
        compiler_params=pltpu.CompilerParams(
            needs_layout_passes=False, use_tc_tiling_on_sc=False
        ),
    )
    def topk_kernel(packed_hbm, topv_hbm, topi_hbm, probs_v, topv_v, topi_v):
        c = lax.axis_index("c")
        s = lax.axis_index("s")
        wid = c * ns + s
        base = wid * rows_per_w
        pltpu.sync_copy(
            packed_hbm.at[pl.ds(s * rows_per_w, rows_per_w), pl.ds(c * _E, _E)],
            probs_v,
        )

        lane = lax.iota(jnp.int32, _LANES)
        in_top = lane < _TOPK
        lane_sh = jnp.maximum(lane - _TOPK, 0)
        lanes_c = [lane + c * _LANES for c in range(_E // _LANES)]

        def merge(ka, va, kb, vb):
            mk = jnp.where(in_top, ka, lax.rev(kb, (0,)))
            mv = jnp.where(in_top, va, lax.rev(vb, (0,)))
            return plsc.sort_key_val(mk, mv, descending=True)

        def one_row(r):
            ks, vs = [], []
            for c in range(_E // _LANES):
                k = probs_v[r, pl.ds(c * _LANES, _LANES)]
                sk, sv = plsc.sort_key_val(k, lanes_c[c], descending=True)
                ks.append(sk)
                vs.append(sv)
            k01, v01 = merge(ks[0], vs[0], ks[1], vs[1])
            k23, v23 = merge(ks[2], vs[2], ks[3], vs[3])
            kt, vt = merge(k01, v01, k23, v23)
            s8 = jnp.sum(jnp.where(in_top, kt, 0.0))
            return kt / s8, vt

        gather_dnums = lax.GatherDimensionNumbers(
            offset_dims=(), collapsed_slice_dims=(0,), start_index_map=(0,)
        )

        def pair_combine(a, b):
            b_sh = lax.gather(
                b,
                lane_sh[:, None],
                gather_dnums,
                (1,),
                mode=lax.GatherScatterMode.PROMISE_IN_BOUNDS,
            )
            return jnp.where(in_top, a, b_sh)

        @plsc.parallel_loop(0, rows_per_w, step=2, unroll=2)
        def _loop(r):
            kv0, vt0 = one_row(r)
            kv1, vt1 = one_row(r + 1)
            topv_v[pl.ds(r * _TOPK, _LANES)] = pair_combine(kv0, kv1)
            topi_v[pl.ds(r * _TOPK, _LANES)] = pair_combine(vt0, vt1)

        pltpu.sync_copy(topv_v, topv_hbm.at[pl.ds(base * _TOPK, rows_per_w * _TOPK)])
        pltpu.sync_copy(topi_v, topi_hbm.at[pl.ds(base * _TOPK, rows_per_w * _TOPK)])

    return topk_kernel


def kernel(hidden_states, weight):
    x = hidden_states.reshape(-1, _H)
    n = x.shape[0]
    packed = _router_probs(x, weight.T)
    topv_flat, topi_flat = _make_topk(n)(packed)
    probs = jnp.concatenate([packed[:, :_E], packed[:, _E:]], axis=0)
    return (
        probs,
        topv_flat.reshape(n, _TOPK),
        topi_flat.reshape(n, _TOPK),
    )


# speedup vs baseline: 1.0615x; 1.0615x over previous
import functools

import jax
import jax.numpy as jnp
from jax import lax
from jax.experimental import pallas as pl
from jax.experimental.pallas import tpu as pltpu
from jax.experimental.pallas import tpu_sc as plsc

_TOPK = 8
_E = 64
_H = 2048
_LANES = 16


def _softmax64(logits):
    m = jnp.max(logits, axis=-1, keepdims=True)
    e = jnp.exp(logits - m)
    return e / jnp.sum(e, axis=-1, keepdims=True)


def _probs_body(xa_ref, xb_ref, wt_ref, probs_ref, packed_ref):
    pa = _softmax64(
        jnp.dot(xa_ref[...], wt_ref[...], preferred_element_type=jnp.float32)
    )
    pb = _softmax64(
        jnp.dot(xb_ref[...], wt_ref[...], preferred_element_type=jnp.float32)
    )
    probs_ref[0] = pa
    probs_ref[1] = pb
    packed_ref[...] = jnp.concatenate([pa, pb], axis=1)


def _router_probs(x, wt, block_rows=1024):
    n = x.shape[0]
    half = n // block_rows // 2
    return pl.pallas_call(
        _probs_body,
        grid=(half,),
        in_specs=[
            pl.BlockSpec((block_rows, _H), lambda i: (i, 0)),
            pl.BlockSpec((block_rows, _H), lambda i: (i + half, 0)),
            pl.BlockSpec((_H, _E), lambda i: (0, 0)),
        ],
        out_specs=[
            pl.BlockSpec((2, block_rows, _E), lambda i: (0, i, 0)),
            pl.BlockSpec((block_rows, 2 * _E), lambda i: (i, 0)),
        ],
        out_shape=[
            jax.ShapeDtypeStruct((2, n // 2, _E), jnp.float32),
            jax.ShapeDtypeStruct((n // 2, 2 * _E), jnp.float32),
        ],
    )(x, x, wt)


def _make_topk(n_rows):
    info = plsc.get_sparse_core_info()
    nc, ns = info.num_cores, info.num_subcores
    nw = nc * ns
    rows_per_w = n_rows // nw

    mesh = plsc.VectorSubcoreMesh(core_axis_name="c", subcore_axis_name="s")

    @functools.partial(
        pl.kernel,
        mesh=mesh,
        out_type=(
            jax.ShapeDtypeStruct((n_rows * _TOPK,), jnp.float32),
            jax.ShapeDtypeStruct((n_rows * _TOPK,), jnp.int32),
        ),
        scratch_types=[
            pltpu.VMEM((rows_per_w, _E), jnp.float32),
            pltpu.VMEM((rows_per_w * _TOPK,), jnp.float32),
            pltpu.VMEM((rows_per_w * _TOPK,), jnp.int32),
        ],
        compiler_params=pltpu.CompilerParams(
            needs_layout_passes=False, use_tc_tiling_on_sc=False
        ),
    )
    def topk_kernel(packed_hbm, topv_hbm, topi_hbm, probs_v, topv_v, topi_v):
        c = lax.axis_index("c")
        s = lax.axis_index("s")
        wid = c * ns + s
        base = wid * rows_per_w
        pltpu.sync_copy(
            packed_hbm.at[pl.ds(s * rows_per_w, rows_per_w), pl.ds(c * _E, _E)],
            probs_v,
        )

        lane = lax.iota(jnp.int32, _LANES)
        in_top = lane < _TOPK
        lane_sh = jnp.maximum(lane - _TOPK, 0)
        lanes_c = [lane + c * _LANES for c in range(_E // _LANES)]

        def merge(ka, va, kb, vb):
            mk = jnp.where(in_top, ka, lax.rev(kb, (0,)))
            mv = jnp.where(in_top, va, lax.rev(vb, (0,)))
            return plsc.sort_key_val(mk, mv, descending=True)

        def one_row(r):
            ks, vs = [], []
            for c in range(_E // _LANES):
                k = probs_v[r, pl.ds(c * _LANES, _LANES)]
                sk, sv = plsc.sort_key_val(k, lanes_c[c], descending=True)
                ks.append(sk)
                vs.append(sv)
            k01, v01 = merge(ks[0], vs[0], ks[1], vs[1])
            k23, v23 = merge(ks[2], vs[2], ks[3], vs[3])
            kt, vt = merge(k01, v01, k23, v23)
            s8 = jnp.sum(jnp.where(in_top, kt, 0.0))
            return kt / s8, vt

        gather_dnums = lax.GatherDimensionNumbers(
            offset_dims=(), collapsed_slice_dims=(0,), start_index_map=(0,)
        )

        def pair_combine(a, b):
            b_sh = lax.gather(
                b,
                lane_sh[:, None],
                gather_dnums,
                (1,),
                mode=lax.GatherScatterMode.PROMISE_IN_BOUNDS,
            )
            return jnp.where(in_top, a, b_sh)

        @plsc.parallel_loop(0, rows_per_w, step=2, unroll=2)
        def _loop(r):
            kv0, vt0 = one_row(r)
            kv1, vt1 = one_row(r + 1)
            topv_v[pl.ds(r * _TOPK, _LANES)] = pair_combine(kv0, kv1)
            topi_v[pl.ds(r * _TOPK, _LANES)] = pair_combine(vt0, vt1)

        pltpu.sync_copy(topv_v, topv_hbm.at[pl.ds(base * _TOPK, rows_per_w * _TOPK)])
        pltpu.sync_copy(topi_v, topi_hbm.at[pl.ds(base * _TOPK, rows_per_w * _TOPK)])

    return topk_kernel


def kernel(hidden_states, weight):
    x = hidden_states.reshape(-1, _H)
    n = x.shape[0]
    probs2, packed = _router_probs(x, weight.T)
    topv_flat, topi_flat = _make_topk(n)(packed)
    return (
        probs2.reshape(n, _E),
        topv_flat.reshape(n, _TOPK),
        topi_flat.reshape(n, _TOPK),
    )


# speedup vs baseline: 1.0622x; 1.0006x over previous
import functools

import jax
import jax.numpy as jnp
from jax import lax
from jax.experimental import pallas as pl
from jax.experimental.pallas import tpu as pltpu
from jax.experimental.pallas import tpu_sc as plsc

_TOPK = 8
_E = 64
_H = 2048
_LANES = 16


def _softmax64(logits):
    m = jnp.max(logits, axis=-1, keepdims=True)
    e = jnp.exp(logits - m)
    return e / jnp.sum(e, axis=-1, keepdims=True)


def _probs_body(xa_ref, xb_ref, wt_ref, probs_ref, packed_ref):
    pa = _softmax64(
        jnp.dot(xa_ref[...], wt_ref[...], preferred_element_type=jnp.float32)
    )
    pb = _softmax64(
        jnp.dot(xb_ref[...], wt_ref[...], preferred_element_type=jnp.float32)
    )
    probs_ref[0] = pa
    probs_ref[1] = pb
    packed_ref[...] = jnp.concatenate([pa, pb], axis=1)


def _router_probs(x, wt, block_rows=1024):
    n = x.shape[0]
    half = n // block_rows // 2
    return pl.pallas_call(
        _probs_body,
        grid=(half,),
        in_specs=[
            pl.BlockSpec((block_rows, _H), lambda i: (i, 0)),
            pl.BlockSpec((block_rows, _H), lambda i: (i + half, 0)),
            pl.BlockSpec((_H, _E), lambda i: (0, 0)),
        ],
        out_specs=[
            pl.BlockSpec((2, block_rows, _E), lambda i: (0, i, 0)),
            pl.BlockSpec((block_rows, 2 * _E), lambda i: (i, 0)),
        ],
        out_shape=[
            jax.ShapeDtypeStruct((2, n // 2, _E), jnp.float32),
            jax.ShapeDtypeStruct((n // 2, 2 * _E), jnp.float32),
        ],
    )(x, x, wt)


def _make_topk(n_rows):
    info = plsc.get_sparse_core_info()
    nc, ns = info.num_cores, info.num_subcores
    nw = nc * ns
    rows_per_w = n_rows // nw

    mesh = plsc.VectorSubcoreMesh(core_axis_name="c", subcore_axis_name="s")

    @functools.partial(
        pl.kernel,
        mesh=mesh,
        out_type=(
            jax.ShapeDtypeStruct((n_rows // 2, _LANES), jnp.float32),
            jax.ShapeDtypeStruct((n_rows // 2, _LANES), jnp.int32),
        ),
        scratch_types=[
            pltpu.VMEM((rows_per_w, _E), jnp.float32),
            pltpu.VMEM((rows_per_w // 2, _LANES), jnp.float32),
            pltpu.VMEM((rows_per_w // 2, _LANES), jnp.int32),
        ],
        compiler_params=pltpu.CompilerParams(
            needs_layout_passes=False, use_tc_tiling_on_sc=False
        ),
    )
    def topk_kernel(packed_hbm, topv_hbm, topi_hbm, probs_v, topv_v, topi_v):
        c = lax.axis_index("c")
        s = lax.axis_index("s")
        wid = c * ns + s
        base = wid * rows_per_w
        pltpu.sync_copy(
            packed_hbm.at[pl.ds(s * rows_per_w, rows_per_w), pl.ds(c * _E, _E)],
            probs_v,
        )

        lane = lax.iota(jnp.int32, _LANES)
        in_top = lane < _TOPK
        lane_sh = jnp.maximum(lane - _TOPK, 0)
        lanes_c = [lane + c * _LANES for c in range(_E // _LANES)]

        def merge(ka, va, kb, vb):
            mk = jnp.where(in_top, ka, lax.rev(kb, (0,)))
            mv = jnp.where(in_top, va, lax.rev(vb, (0,)))
            return plsc.sort_key_val(mk, mv, descending=True)

        def one_row(r):
            ks, vs = [], []
            for c in range(_E // _LANES):
                k = probs_v[r, pl.ds(c * _LANES, _LANES)]
                sk, sv = plsc.sort_key_val(k, lanes_c[c], descending=True)
                ks.append(sk)
                vs.append(sv)
            k01, v01 = merge(ks[0], vs[0], ks[1], vs[1])
            k23, v23 = merge(ks[2], vs[2], ks[3], vs[3])
            kt, vt = merge(k01, v01, k23, v23)
            s8 = jnp.sum(jnp.where(in_top, kt, 0.0))
            return kt / s8, vt

        gather_dnums = lax.GatherDimensionNumbers(
            offset_dims=(), collapsed_slice_dims=(0,), start_index_map=(0,)
        )

        def pair_combine(a, b):
            b_sh = lax.gather(
                b,
                lane_sh[:, None],
                gather_dnums,
                (1,),
                mode=lax.GatherScatterMode.PROMISE_IN_BOUNDS,
            )
            return jnp.where(in_top, a, b_sh)

        @plsc.parallel_loop(0, rows_per_w, step=2, unroll=2)
        def _loop(r):
            kv0, vt0 = one_row(r)
            kv1, vt1 = one_row(r + 1)
            p = lax.div(r, 2)
            topv_v[p] = pair_combine(kv0, kv1)
            topi_v[p] = pair_combine(vt0, vt1)

        pair_base = base // 2
        pairs_per_w = rows_per_w // 2
        pltpu.sync_copy(topv_v, topv_hbm.at[pl.ds(pair_base, pairs_per_w)])
        pltpu.sync_copy(topi_v, topi_hbm.at[pl.ds(pair_base, pairs_per_w)])

    return topk_kernel


def kernel(hidden_states, weight):
    x = hidden_states.reshape(-1, _H)
    n = x.shape[0]
    probs2, packed = _router_probs(x, weight.T)
    topv_pairs, topi_pairs = _make_topk(n)(packed)
    return (
        probs2.reshape(n, _E),
        topv_pairs.reshape(n, _TOPK),
        topi_pairs.reshape(n, _TOPK),
    )
